# Initial kernel scaffold; baseline (speedup 1.0000x reference)
import jax
import jax.numpy as jnp
from jax import lax
from jax.experimental import pallas as pl
from jax.experimental.pallas import tpu as pltpu
from jax.experimental.pallas import tpu_sc as plsc

N_USERS = 20000
N_ITEMS = 30000
N_NODES = N_USERS + N_ITEMS
DIM = 64
HDIM = DIM // 2
N_LAYERS = 3
N_EDGES = 800000

L = 16
CHUNK = 128
N_CHUNKS = N_EDGES // CHUNK
N_SUB = 16
ROWS_BLK = 125
N_BLKS = N_NODES // ROWS_BLK


def _layer_body(emb_lo, emb_hi, src, dst, w, out_lo, out_hi,
                sidx, dstb, wb, msgs, stage, acc):
    sc = lax.axis_index("c")
    sub = lax.axis_index("s")

    def half(emb_ref, out_ref):
        @pl.loop(0, ROWS_BLK)
        def _(r):
            stage[r, pl.ds(0, L)] = jnp.zeros((L,), jnp.float32)
            stage[r, pl.ds(L, L)] = jnp.zeros((L,), jnp.float32)

        @pl.loop(sub, N_BLKS, step=N_SUB)
        def _(b):
            pltpu.sync_copy(stage, acc.at[pl.ds(b * ROWS_BLK, ROWS_BLK)])

        plsc.subcore_barrier()

        @pl.loop(sub, N_CHUNKS, step=N_SUB)
        def _(c):
            base = c * CHUNK
            pltpu.sync_copy(src.at[pl.ds(base, CHUNK)], sidx)
            pltpu.sync_copy(dst.at[pl.ds(base, CHUNK)], dstb)
            pltpu.sync_copy(w.at[pl.ds(base, CHUNK)], wb)
            pltpu.sync_copy(emb_ref.at[sidx], msgs)

            @pl.loop(0, CHUNK)
            def _(e):
                wv = jnp.full((L,), wb[e], jnp.float32)
                msgs[e, pl.ds(0, L)] = msgs[e, pl.ds(0, L)] * wv
                msgs[e, pl.ds(L, L)] = msgs[e, pl.ds(L, L)] * wv

            pltpu.sync_copy(msgs, acc.at[dstb], add=True)

        plsc.subcore_barrier()

        @pl.loop(sub, N_BLKS, step=N_SUB)
        def _(b):
            r0 = b * ROWS_BLK
            pltpu.sync_copy(acc.at[pl.ds(r0, ROWS_BLK)], stage)
            pltpu.sync_copy(stage, out_ref.at[pl.ds(r0, ROWS_BLK)])

    @pl.when(sc == 0)
    def _():
        half(emb_lo, out_lo)

    @pl.when(sc == 1)
    def _():
        half(emb_hi, out_hi)


@jax.jit
def _layer(emb_lo, emb_hi, src, dst, w):
    mesh = plsc.VectorSubcoreMesh(core_axis_name="c", subcore_axis_name="s")
    f = pl.kernel(
        _layer_body,
        out_type=(
            jax.ShapeDtypeStruct((N_NODES, HDIM), jnp.float32),
            jax.ShapeDtypeStruct((N_NODES, HDIM), jnp.float32),
        ),
        mesh=mesh,
        scratch_types=[
            pltpu.VMEM((CHUNK,), jnp.int32),
            pltpu.VMEM((CHUNK,), jnp.int32),
            pltpu.VMEM((CHUNK,), jnp.float32),
            pltpu.VMEM((CHUNK, HDIM), jnp.float32),
            pltpu.VMEM((ROWS_BLK, HDIM), jnp.float32),
            pltpu.VMEM_SHARED((N_NODES, HDIM), jnp.float32),
        ],
    )
    return f(emb_lo, emb_hi, src, dst, w)


def kernel(user_emb, item_emb, edge_index, edge_weight):
    all0 = jnp.concatenate([user_emb.astype(jnp.float32),
                            item_emb.astype(jnp.float32)], axis=0)
    src = edge_index[0].astype(jnp.int32)
    dst = edge_index[1].astype(jnp.int32)
    w = edge_weight.astype(jnp.float32)

    lo, hi = all0[:, :HDIM], all0[:, HDIM:]
    halves = [(lo, hi)]
    for _ in range(N_LAYERS):
        lo, hi = _layer(lo, hi, src, dst, w)
        halves.append((lo, hi))

    embs = jnp.stack([jnp.concatenate(p, axis=-1) for p in halves], axis=1)
    return embs[:N_USERS], embs[N_USERS:]


# speedup vs baseline: 3.4123x; 3.4123x over previous
import jax
import jax.numpy as jnp
from jax import lax
from jax._src import config as _jax_config
from jax.experimental import pallas as pl
from jax.experimental.pallas import tpu as pltpu
from jax.experimental.pallas import tpu_sc as plsc

N_USERS = 20000
N_ITEMS = 30000
N_NODES = N_USERS + N_ITEMS
DIM = 64
HDIM = DIM // 2
N_LAYERS = 3
N_EDGES = 800000

L = 16
CHUNK = 128
N_CHUNKS = N_EDGES // CHUNK
N_SUB = 16
ROWS_BLK = 200
N_BLKS = N_NODES // ROWS_BLK


def _layer_body(emb_lo, emb_hi, src, dst, w, out_lo, out_hi,
                sidx, dstb, wb, msgs, stage, acc):
    sc = lax.axis_index("c")
    sub = lax.axis_index("s")

    def half(emb_ref, out_ref):
        i32 = jnp.int32
        @pl.loop(0, ROWS_BLK)
        def _(r):
            stage[r, pl.ds(0, L)] = jnp.zeros((L,), jnp.float32)
            stage[r, pl.ds(L, L)] = jnp.zeros((L,), jnp.float32)

        @pl.loop(sub, N_BLKS, step=N_SUB)
        def _(b):
            pltpu.sync_copy(stage, acc.at[pl.ds(b * i32(ROWS_BLK), ROWS_BLK)])

        plsc.subcore_barrier()

        @pl.loop(sub, N_CHUNKS, step=N_SUB)
        def _(c):
            base = c * i32(CHUNK)
            pltpu.sync_copy(src.at[pl.ds(base, CHUNK)], sidx)
            pltpu.sync_copy(dst.at[pl.ds(base, CHUNK)], dstb)
            pltpu.sync_copy(w.at[pl.ds(base, CHUNK)], wb)
            pltpu.sync_copy(emb_ref.at[sidx], msgs)

            @pl.loop(0, CHUNK, step=L)
            def _(e0):
                wg = wb[pl.ds(e0, L)]
                for j in range(L):
                    wv = jnp.full((L,), wg[j], jnp.float32)
                    e = e0 + i32(j)
                    msgs[e, pl.ds(0, L)] = msgs[e, pl.ds(0, L)] * wv
                    msgs[e, pl.ds(L, L)] = msgs[e, pl.ds(L, L)] * wv

            pltpu.sync_copy(msgs, acc.at[dstb], add=True)

        plsc.subcore_barrier()

        @pl.loop(sub, N_BLKS, step=N_SUB)
        def _(b):
            r0 = b * i32(ROWS_BLK)
            pltpu.sync_copy(acc.at[pl.ds(r0, ROWS_BLK)], stage)
            pltpu.sync_copy(stage, out_ref.at[pl.ds(r0, ROWS_BLK)])

    @pl.when(sc == 0)
    def _():
        half(emb_lo, out_lo)

    @pl.when(sc == 1)
    def _():
        half(emb_hi, out_hi)


@jax.jit
def _layer(emb_lo, emb_hi, src, dst, w):
    mesh = plsc.VectorSubcoreMesh(core_axis_name="c", subcore_axis_name="s")
    f = pl.kernel(
        _layer_body,
        out_type=(
            jax.ShapeDtypeStruct((N_NODES, HDIM), jnp.float32),
            jax.ShapeDtypeStruct((N_NODES, HDIM), jnp.float32),
        ),
        mesh=mesh,
        scratch_types=[
            pltpu.VMEM((CHUNK,), jnp.int32),
            pltpu.VMEM((CHUNK,), jnp.int32),
            pltpu.VMEM((CHUNK,), jnp.float32),
            pltpu.VMEM((CHUNK, HDIM), jnp.float32),
            pltpu.VMEM((ROWS_BLK, HDIM), jnp.float32),
            pltpu.VMEM_SHARED((N_NODES, HDIM), jnp.float32),
        ],
        compiler_params=pltpu.CompilerParams(use_tc_tiling_on_sc=False),
    )
    return f(emb_lo, emb_hi, src, dst, w)


def kernel(user_emb, item_emb, edge_index, edge_weight):
    with _jax_config.enable_x64(False):
        return _kernel_32(user_emb, item_emb, edge_index, edge_weight)


def _kernel_32(user_emb, item_emb, edge_index, edge_weight):
    all0 = jnp.concatenate([user_emb.astype(jnp.float32),
                            item_emb.astype(jnp.float32)], axis=0)
    src = edge_index[0].astype(jnp.int32)
    dst = edge_index[1].astype(jnp.int32)
    w = edge_weight.astype(jnp.float32)

    lo, hi = all0[:, :HDIM], all0[:, HDIM:]
    halves = [(lo, hi)]
    for _ in range(N_LAYERS):
        lo, hi = _layer(lo, hi, src, dst, w)
        halves.append((lo, hi))

    embs = jnp.stack([jnp.concatenate(p, axis=-1) for p in halves], axis=1)
    return embs[:N_USERS], embs[N_USERS:]


# speedup vs baseline: 6.7901x; 1.9899x over previous
import jax
import jax.numpy as jnp
from jax import lax
from jax._src import config as _jax_config
from jax.experimental import pallas as pl
from jax.experimental.pallas import tpu as pltpu
from jax.experimental.pallas import tpu_sc as plsc

N_USERS = 20000
N_ITEMS = 30000
N_NODES = N_USERS + N_ITEMS
DIM = 64
HDIM = DIM // 2
N_LAYERS = 3
N_EDGES = 800000

L = 16
CHUNK = 128
SB = 3
N_SUB = 16
SUPS_PER_TILE = 132
N_SUPS = N_SUB * SUPS_PER_TILE
E_PAD = N_SUPS * SB * CHUNK
ROWS_BLK = 100
N_BLKS = N_NODES // ROWS_BLK


def _layer_body(emb_lo, emb_hi, src2, dst2, w2, out_lo, out_hi,
                sidx0, sidx1, dstb0, dstb1, wb0, wb1, msgs0, msgs1,
                stage, acc, gsem0, gsem1, ssem0, ssem1):
    sc = lax.axis_index("c")
    sub = lax.axis_index("s")
    i32 = jnp.int32

    def half(emb_ref, out_ref):
        @pl.loop(0, ROWS_BLK)
        def _(r):
            stage[r, pl.ds(0, L)] = jnp.zeros((L,), jnp.float32)
            stage[r, pl.ds(L, L)] = jnp.zeros((L,), jnp.float32)

        @pl.loop(sub, N_BLKS, step=N_SUB)
        def _(b):
            pltpu.sync_copy(stage, acc.at[pl.ds(b * i32(ROWS_BLK), ROWS_BLK)])

        plsc.subcore_barrier()

        def load_idx(sb_, db_, wb_, s):
            r = s * i32(SB)
            pltpu.sync_copy(src2.at[pl.ds(r, SB)], sb_)
            pltpu.sync_copy(dst2.at[pl.ds(r, SB)], db_)
            pltpu.sync_copy(w2.at[pl.ds(r, SB)], wb_)

        def fire_g(sb_, mb_, sem):
            for j in range(SB):
                pltpu.async_copy(emb_ref.at[sb_.at[j]], mb_.at[j], sem)

        def wait_g(sb_, mb_, sem):
            for j in range(SB):
                pltpu.make_async_copy(emb_ref.at[sb_.at[j]], mb_.at[j],
                                      sem).wait()

        def fire_s(db_, mb_, sem):
            for j in range(SB):
                pltpu.async_copy(mb_.at[j], acc.at[db_.at[j]], sem, add=True)

        def wait_s(db_, mb_, sem):
            for j in range(SB):
                pltpu.make_async_copy(mb_.at[j], acc.at[db_.at[j]],
                                      sem).wait()

        def compute(wb_, mb_):
            @pl.loop(0, SB)
            def _(c):
                @pl.loop(0, CHUNK, step=L)
                def _(e0):
                    wg = wb_[c, pl.ds(e0, L)]
                    for j in range(L):
                        wv = jnp.full((L,), wg[j], jnp.float32)
                        e = e0 + i32(j)
                        mb_[c, e, pl.ds(0, L)] = mb_[c, e, pl.ds(0, L)] * wv
                        mb_[c, e, pl.ds(L, L)] = mb_[c, e, pl.ds(L, L)] * wv

        def sup(i):
            return sub + i32(N_SUB) * i

        load_idx(sidx0, dstb0, wb0, sup(i32(0)))
        fire_g(sidx0, msgs0, gsem0)
        load_idx(sidx1, dstb1, wb1, sup(i32(1)))
        fire_g(sidx1, msgs1, gsem1)

        @pl.loop(0, (SUPS_PER_TILE - 2) // 2)
        def _(k):
            i0 = i32(2) * k + i32(2)
            wait_g(sidx0, msgs0, gsem0)
            compute(wb0, msgs0)
            fire_s(dstb0, msgs0, ssem0)
            wait_g(sidx1, msgs1, gsem1)
            compute(wb1, msgs1)
            fire_s(dstb1, msgs1, ssem1)
            wait_s(dstb0, msgs0, ssem0)
            load_idx(sidx0, dstb0, wb0, sup(i0))
            fire_g(sidx0, msgs0, gsem0)
            wait_s(dstb1, msgs1, ssem1)
            load_idx(sidx1, dstb1, wb1, sup(i0 + i32(1)))
            fire_g(sidx1, msgs1, gsem1)

        wait_g(sidx0, msgs0, gsem0)
        compute(wb0, msgs0)
        fire_s(dstb0, msgs0, ssem0)
        wait_g(sidx1, msgs1, gsem1)
        compute(wb1, msgs1)
        fire_s(dstb1, msgs1, ssem1)
        wait_s(dstb0, msgs0, ssem0)
        wait_s(dstb1, msgs1, ssem1)

        plsc.subcore_barrier()

        @pl.loop(sub, N_BLKS, step=N_SUB)
        def _(b):
            r0 = b * i32(ROWS_BLK)
            pltpu.sync_copy(acc.at[pl.ds(r0, ROWS_BLK)], stage)
            pltpu.sync_copy(stage, out_ref.at[pl.ds(r0, ROWS_BLK)])

    @pl.when(sc == 0)
    def _():
        half(emb_lo, out_lo)

    @pl.when(sc == 1)
    def _():
        half(emb_hi, out_hi)


@jax.jit
def _layer(emb_lo, emb_hi, src2, dst2, w2):
    mesh = plsc.VectorSubcoreMesh(core_axis_name="c", subcore_axis_name="s")
    f = pl.kernel(
        _layer_body,
        out_type=(
            jax.ShapeDtypeStruct((N_NODES, HDIM), jnp.float32),
            jax.ShapeDtypeStruct((N_NODES, HDIM), jnp.float32),
        ),
        mesh=mesh,
        scratch_types=[
            pltpu.VMEM((SB, CHUNK), jnp.int32),
            pltpu.VMEM((SB, CHUNK), jnp.int32),
            pltpu.VMEM((SB, CHUNK), jnp.int32),
            pltpu.VMEM((SB, CHUNK), jnp.int32),
            pltpu.VMEM((SB, CHUNK), jnp.float32),
            pltpu.VMEM((SB, CHUNK), jnp.float32),
            pltpu.VMEM((SB, CHUNK, HDIM), jnp.float32),
            pltpu.VMEM((SB, CHUNK, HDIM), jnp.float32),
            pltpu.VMEM((ROWS_BLK, HDIM), jnp.float32),
            pltpu.VMEM_SHARED((N_NODES, HDIM), jnp.float32),
            pltpu.SemaphoreType.DMA,
            pltpu.SemaphoreType.DMA,
            pltpu.SemaphoreType.DMA,
            pltpu.SemaphoreType.DMA,
        ],
        compiler_params=pltpu.CompilerParams(use_tc_tiling_on_sc=False),
    )
    return f(emb_lo, emb_hi, src2, dst2, w2)


def kernel(user_emb, item_emb, edge_index, edge_weight):
    with _jax_config.enable_x64(False):
        return _kernel_32(user_emb, item_emb, edge_index, edge_weight)


def _kernel_32(user_emb, item_emb, edge_index, edge_weight):
    all0 = jnp.concatenate([user_emb.astype(jnp.float32),
                            item_emb.astype(jnp.float32)], axis=0)
    pad = E_PAD - N_EDGES
    src2 = jnp.pad(edge_index[0].astype(jnp.int32), (0, pad)).reshape(-1, CHUNK)
    dst2 = jnp.pad(edge_index[1].astype(jnp.int32), (0, pad)).reshape(-1, CHUNK)
    w2 = jnp.pad(edge_weight.astype(jnp.float32), (0, pad)).reshape(-1, CHUNK)

    lo, hi = all0[:, :HDIM], all0[:, HDIM:]
    halves = [(lo, hi)]
    for _ in range(N_LAYERS):
        lo, hi = _layer(lo, hi, src2, dst2, w2)
        halves.append((lo, hi))

    embs = jnp.stack([jnp.concatenate(p, axis=-1) for p in halves], axis=1)
    return embs[:N_USERS], embs[N_USERS:]
